# baseline (speedup 1.0000x reference)
import functools

import jax
import jax.numpy as jnp
from jax import lax
from jax.experimental import pallas as pl
from jax.experimental.pallas import tpu as pltpu
from jax.experimental.pallas import tpu_sc as plsc

NC = 2
NS = 16
NW = NC * NS
LANES = 16
SLOTS = 126


def _prepack_body(off_c, off_n, obs_ref, coords_ref, n_ref, out_ref):
    cb = out_ref.shape[0]
    o = obs_ref[...].reshape(cb, -1)
    c = coords_ref[...] * 2 + off_c
    out_ref[...] = jnp.concatenate(
        [c[:, 0:1], c[:, 0:1] + 1, c[:, 1:2], c[:, 1:2] + 1,
         o, n_ref[...] + off_n, o[:, 0:2]], axis=1)


def _sc_body(C, comb_hbm, idx_hbm, out_hbm, obsidx, outbuf, sem):
    wid = lax.axis_index("s") * NC + lax.axis_index("c")
    B = out_hbm.shape[0]
    rows_per = B // NW
    nch = rows_per // C

    @pl.loop(0, nch)
    def _chunk(g):
        r0 = wid * rows_per + g * C
        pltpu.sync_copy(idx_hbm.at[pl.ds(r0, C)], obsidx)
        cps = [pltpu.async_copy(comb_hbm.at[obsidx.at[i]],
                                outbuf.at[i], sem)
               for i in range(C)]
        for cp in cps:
            cp.wait()
        pltpu.sync_copy(outbuf.at[:, pl.ds(0, SLOTS)],
                        out_hbm.at[pl.ds(r0, C)])


def kernel(coords, obses, n_completed, coord_table, field_table,
           completed_table):
    B = coords.shape[0]
    coords = coords.astype(jnp.int32)
    obses = obses.astype(jnp.int32)
    n_completed = n_completed.astype(jnp.int32)
    fdim = field_table.shape[1]
    off_n = field_table.shape[0]
    off_c = off_n + completed_table.shape[0]
    comb = jnp.concatenate(
        [field_table, completed_table, coord_table.reshape(-1, fdim)], axis=0)

    CB = 2048
    idxrows = pl.pallas_call(
        functools.partial(_prepack_body, off_c, off_n),
        out_shape=jax.ShapeDtypeStruct((B, 128), jnp.int32),
        grid=(B // CB,),
        in_specs=[
            pl.BlockSpec((CB, obses.shape[1], obses.shape[2]),
                         lambda i: (i, 0, 0)),
            pl.BlockSpec((CB, 2), lambda i: (i, 0)),
            pl.BlockSpec((CB, 1), lambda i: (i, 0)),
        ],
        out_specs=pl.BlockSpec((CB, 128), lambda i: (i, 0)),
    )(obses, coords, n_completed)

    C = 32
    mesh = plsc.VectorSubcoreMesh(core_axis_name="c", subcore_axis_name="s")
    out = pl.kernel(
        functools.partial(_sc_body, C),
        out_type=jax.ShapeDtypeStruct((B, SLOTS, fdim), jnp.float32),
        mesh=mesh,
        compiler_params=pltpu.CompilerParams(
            use_tc_tiling_on_sc=False,
            needs_layout_passes=False,
        ),
        scratch_types=[
            pltpu.VMEM((C, 128), jnp.int32),
            pltpu.VMEM((C, 128, fdim), jnp.float32),
            pltpu.SemaphoreType.DMA,
        ],
    )(comb, idxrows)
    return out.reshape(B, SLOTS * fdim)


# speedup vs baseline: 1.0039x; 1.0039x over previous
import functools

import jax
import jax.numpy as jnp
from jax import lax
from jax.experimental import pallas as pl
from jax.experimental.pallas import tpu as pltpu
from jax.experimental.pallas import tpu_sc as plsc

NC = 2
NS = 16
NW = NC * NS
LANES = 16
SLOTS = 126


def _prepack_body(off_c, off_n, obs_ref, coords_ref, n_ref, out_ref):
    cb = out_ref.shape[0]
    o = obs_ref[...].reshape(cb, -1)
    c = coords_ref[...] * 2 + off_c
    out_ref[...] = jnp.concatenate(
        [c[:, 0:1], c[:, 0:1] + 1, c[:, 1:2], c[:, 1:2] + 1,
         o, n_ref[...] + off_n, o[:, 0:2]], axis=1)


def _sc_body(C, comb_hbm, idx_hbm, out_hbm, obsidx, outbuf, sem):
    wid = lax.axis_index("s") * NC + lax.axis_index("c")
    B = out_hbm.shape[0]
    rows_per = B // NW
    nch = rows_per // C

    @pl.loop(0, nch)
    def _chunk(g):
        r0 = wid * rows_per + g * C
        pltpu.sync_copy(idx_hbm.at[pl.ds(r0, C)], obsidx)
        cps = [pltpu.async_copy(comb_hbm.at[obsidx.at[i]],
                                outbuf.at[i], sem)
               for i in range(C)]
        for cp in cps:
            cp.wait()
        pltpu.sync_copy(outbuf, out_hbm.at[pl.ds(r0, C)])


def kernel(coords, obses, n_completed, coord_table, field_table,
           completed_table):
    B = coords.shape[0]
    coords = coords.astype(jnp.int32)
    obses = obses.astype(jnp.int32)
    n_completed = n_completed.astype(jnp.int32)
    fdim = field_table.shape[1]
    off_n = field_table.shape[0]
    off_c = off_n + completed_table.shape[0]
    comb = jnp.concatenate(
        [field_table, completed_table, coord_table.reshape(-1, fdim)], axis=0)

    CB = 2048
    idxrows = pl.pallas_call(
        functools.partial(_prepack_body, off_c, off_n),
        out_shape=jax.ShapeDtypeStruct((B, 128), jnp.int32),
        grid=(B // CB,),
        in_specs=[
            pl.BlockSpec((CB, obses.shape[1], obses.shape[2]),
                         lambda i: (i, 0, 0)),
            pl.BlockSpec((CB, 2), lambda i: (i, 0)),
            pl.BlockSpec((CB, 1), lambda i: (i, 0)),
        ],
        out_specs=pl.BlockSpec((CB, 128), lambda i: (i, 0)),
    )(obses, coords, n_completed)

    C = 32
    mesh = plsc.VectorSubcoreMesh(core_axis_name="c", subcore_axis_name="s")
    out = pl.kernel(
        functools.partial(_sc_body, C),
        out_type=jax.ShapeDtypeStruct((B, 128, fdim), jnp.float32),
        mesh=mesh,
        compiler_params=pltpu.CompilerParams(
            use_tc_tiling_on_sc=False,
            needs_layout_passes=False,
        ),
        scratch_types=[
            pltpu.VMEM((C, 128), jnp.int32),
            pltpu.VMEM((C, 128, fdim), jnp.float32),
            pltpu.SemaphoreType.DMA,
        ],
    )(comb, idxrows)
    return out[:, :SLOTS, :].reshape(B, SLOTS * fdim)
